# Initial kernel scaffold; baseline (speedup 1.0000x reference)
import jax
import jax.numpy as jnp
from jax.experimental import pallas as pl


def kernel(grid):
    raise NotImplementedError("write your pallas kernel here")


# speedup vs baseline: 684.9092x; 684.9092x over previous
import functools

import jax
import jax.numpy as jnp
import numpy as np
from jax import lax
from jax.experimental import pallas as pl
from jax.experimental.pallas import tpu as pltpu
from jax.experimental.pallas import tpu_sc as plsc

GRID_CELLS = 81
LANES = 16

_ENT_TABLE = np.array(
    [np.log2(max(9 - bin(m).count("1"), 1)) for m in range(512)],
    dtype=np.float32,
)


def _entropy_body(grid_hbm, table_hbm, out_hbm, chunk_v, table_v, out_v,
                  *, elems_per_worker, num_cores):
    wid = lax.axis_index("s") * num_cores + lax.axis_index("c")
    chunk = elems_per_worker * GRID_CELLS
    base = wid * chunk
    pltpu.sync_copy(grid_hbm.at[pl.ds(base, chunk)], chunk_v)
    pltpu.sync_copy(table_hbm, table_v)

    lane = lax.iota(jnp.int32, LANES)
    num_groups = elems_per_worker // LANES

    def group_body(g, carry):
        idx0 = (g * LANES + lane) * GRID_CELLS

        rowm = [None] * 9
        colm = [None] * 9
        boxm = [None] * 9

        def acc_or(lst, k, b):
            lst[k] = b if lst[k] is None else lst[k] | b

        for r in range(9):
            for c in range(9):
                vf = plsc.load_gather(chunk_v, [idx0 + (r * 9 + c)])
                vi = vf.astype(jnp.int32)
                b = (jnp.full((LANES,), 1, jnp.int32) << vi) >> 1
                acc_or(rowm, r, b)
                acc_or(colm, c, b)
                acc_or(boxm, (r // 3) * 3 + (c // 3), b)

        acc = jnp.zeros((LANES,), jnp.float32)
        for r in range(9):
            for c in range(9):
                m = rowm[r] | colm[c] | boxm[(r // 3) * 3 + (c // 3)]
                ent = plsc.load_gather(table_v, [m])
                vf = plsc.load_gather(chunk_v, [idx0 + (r * 9 + c)])
                acc = acc + jnp.where(vf == 0.0, ent, 0.0)
        out_v[pl.ds(g * LANES, LANES)] = acc
        return carry

    lax.fori_loop(0, num_groups, group_body, 0)
    pltpu.sync_copy(out_v, out_hbm.at[pl.ds(wid * elems_per_worker,
                                            elems_per_worker)])


def kernel(grid):
    batch = grid.shape[0]
    info = plsc.get_sparse_core_info()
    num_cores, num_subcores = info.num_cores, info.num_subcores
    num_workers = num_cores * num_subcores
    elems_per_worker = batch // num_workers
    assert batch % (num_workers * LANES) == 0

    entropy_kernel = pl.kernel(
        functools.partial(
            _entropy_body,
            elems_per_worker=elems_per_worker,
            num_cores=num_cores,
        ),
        mesh=plsc.VectorSubcoreMesh(core_axis_name="c", subcore_axis_name="s"),
        compiler_params=pltpu.CompilerParams(
            use_tc_tiling_on_sc=False,
            needs_layout_passes=False,
        ),
        out_type=jax.ShapeDtypeStruct((batch,), jnp.float32),
        scratch_types=[
            pltpu.VMEM((elems_per_worker * GRID_CELLS,), jnp.float32),
            pltpu.VMEM((512,), jnp.float32),
            pltpu.VMEM((elems_per_worker,), jnp.float32),
        ],
    )
    flat = grid.reshape(-1)
    return entropy_kernel(flat, jnp.asarray(_ENT_TABLE))


# speedup vs baseline: 694.7681x; 1.0144x over previous
import functools

import jax
import jax.numpy as jnp
import numpy as np
from jax import lax
from jax.experimental import pallas as pl
from jax.experimental.pallas import tpu as pltpu
from jax.experimental.pallas import tpu_sc as plsc

GRID_CELLS = 81
LANES = 16

_ENT_TABLE = np.array(
    [np.log2(max(9 - bin(m).count("1"), 1)) for m in range(512)],
    dtype=np.float32,
)


def _entropy_body(grid_hbm, table_hbm, out_hbm, chunk_v, table_v, out_v,
                  *, elems_per_worker, num_cores):
    wid = lax.axis_index("s") * num_cores + lax.axis_index("c")
    chunk = elems_per_worker * GRID_CELLS
    base = wid * chunk
    pltpu.sync_copy(grid_hbm.at[pl.ds(base, chunk)], chunk_v)
    pltpu.sync_copy(table_hbm, table_v)

    lane = lax.iota(jnp.int32, LANES)
    num_groups = elems_per_worker // LANES

    @plsc.parallel_loop(0, num_groups)
    def group_body(g):
        idx0 = (g * LANES + lane) * GRID_CELLS

        rowm = [None] * 9
        colm = [None] * 9
        boxm = [None] * 9

        def acc_or(lst, k, b):
            lst[k] = b if lst[k] is None else lst[k] | b

        for r in range(9):
            for c in range(9):
                vf = plsc.load_gather(chunk_v, [idx0 + (r * 9 + c)])
                vi = vf.astype(jnp.int32)
                b = (jnp.full((LANES,), 1, jnp.int32) << vi) >> 1
                acc_or(rowm, r, b)
                acc_or(colm, c, b)
                acc_or(boxm, (r // 3) * 3 + (c // 3), b)

        rb = [[rowm[r] | boxm[(r // 3) * 3 + bc] for bc in range(3)]
              for r in range(9)]

        accs = [jnp.zeros((LANES,), jnp.float32) for _ in range(3)]
        for r in range(9):
            for c in range(9):
                m = rb[r][c // 3] | colm[c]
                ent = plsc.load_gather(table_v, [m])
                vf = plsc.load_gather(chunk_v, [idx0 + (r * 9 + c)])
                accs[c % 3] = accs[c % 3] + jnp.where(vf == 0.0, ent, 0.0)
        out_v[pl.ds(g * LANES, LANES)] = accs[0] + accs[1] + accs[2]
    pltpu.sync_copy(out_v, out_hbm.at[pl.ds(wid * elems_per_worker,
                                            elems_per_worker)])


def kernel(grid):
    batch = grid.shape[0]
    info = plsc.get_sparse_core_info()
    num_cores, num_subcores = info.num_cores, info.num_subcores
    num_workers = num_cores * num_subcores
    elems_per_worker = batch // num_workers
    assert batch % (num_workers * LANES) == 0

    entropy_kernel = pl.kernel(
        functools.partial(
            _entropy_body,
            elems_per_worker=elems_per_worker,
            num_cores=num_cores,
        ),
        mesh=plsc.VectorSubcoreMesh(core_axis_name="c", subcore_axis_name="s"),
        compiler_params=pltpu.CompilerParams(
            use_tc_tiling_on_sc=False,
            needs_layout_passes=False,
        ),
        out_type=jax.ShapeDtypeStruct((batch,), jnp.float32),
        scratch_types=[
            pltpu.VMEM((elems_per_worker * GRID_CELLS,), jnp.float32),
            pltpu.VMEM((512,), jnp.float32),
            pltpu.VMEM((elems_per_worker,), jnp.float32),
        ],
    )
    flat = grid.reshape(-1)
    return entropy_kernel(flat, jnp.asarray(_ENT_TABLE))


# speedup vs baseline: 697.3877x; 1.0038x over previous
import functools

import jax
import jax.numpy as jnp
import numpy as np
from jax import lax
from jax.experimental import pallas as pl
from jax.experimental.pallas import tpu as pltpu
from jax.experimental.pallas import tpu_sc as plsc

GRID_CELLS = 81
LANES = 16

_ENT_TABLE = np.array(
    [np.log2(max(9 - bin(m).count("1"), 1)) for m in range(512)],
    dtype=np.float32,
)


def _entropy_body(grid_hbm, table_hbm, out_hbm, chunk_v, table_v, out_v,
                  sem0, sem1, *, elems_per_worker, num_cores):
    wid = lax.axis_index("s") * num_cores + lax.axis_index("c")
    chunk = elems_per_worker * GRID_CELLS
    half = chunk // 2
    base = wid * chunk
    cp0 = pltpu.async_copy(grid_hbm.at[pl.ds(base, half)],
                           chunk_v.at[pl.ds(0, half)], sem0)
    cp1 = pltpu.async_copy(grid_hbm.at[pl.ds(base + half, half)],
                           chunk_v.at[pl.ds(half, half)], sem1)
    pltpu.sync_copy(table_hbm, table_v)

    lane = lax.iota(jnp.int32, LANES)
    num_groups = elems_per_worker // LANES

    def group_body(g):
        idx0 = (g * LANES + lane) * GRID_CELLS

        rowm = [None] * 9
        colm = [None] * 9
        boxm = [None] * 9

        def acc_or(lst, k, b):
            lst[k] = b if lst[k] is None else lst[k] | b

        for r in range(9):
            for c in range(9):
                vf = plsc.load_gather(chunk_v, [idx0 + (r * 9 + c)])
                vi = vf.astype(jnp.int32)
                b = (jnp.full((LANES,), 1, jnp.int32) << vi) >> 1
                acc_or(rowm, r, b)
                acc_or(colm, c, b)
                acc_or(boxm, (r // 3) * 3 + (c // 3), b)

        rb = [[rowm[r] | boxm[(r // 3) * 3 + bc] for bc in range(3)]
              for r in range(9)]

        accs = [jnp.zeros((LANES,), jnp.float32) for _ in range(3)]
        for r in range(9):
            for c in range(9):
                m = rb[r][c // 3] | colm[c]
                ent = plsc.load_gather(table_v, [m])
                vf = plsc.load_gather(chunk_v, [idx0 + (r * 9 + c)])
                accs[c % 3] = accs[c % 3] + jnp.where(vf == 0.0, ent, 0.0)
        out_v[pl.ds(g * LANES, LANES)] = accs[0] + accs[1] + accs[2]

    cp0.wait()
    plsc.parallel_loop(0, num_groups // 2)(group_body)
    cp1.wait()
    plsc.parallel_loop(num_groups // 2, num_groups)(group_body)
    pltpu.sync_copy(out_v, out_hbm.at[pl.ds(wid * elems_per_worker,
                                            elems_per_worker)])


def kernel(grid):
    batch = grid.shape[0]
    info = plsc.get_sparse_core_info()
    num_cores, num_subcores = info.num_cores, info.num_subcores
    num_workers = num_cores * num_subcores
    elems_per_worker = batch // num_workers
    assert batch % (num_workers * LANES) == 0

    entropy_kernel = pl.kernel(
        functools.partial(
            _entropy_body,
            elems_per_worker=elems_per_worker,
            num_cores=num_cores,
        ),
        mesh=plsc.VectorSubcoreMesh(core_axis_name="c", subcore_axis_name="s"),
        compiler_params=pltpu.CompilerParams(
            use_tc_tiling_on_sc=False,
            needs_layout_passes=False,
        ),
        out_type=jax.ShapeDtypeStruct((batch,), jnp.float32),
        scratch_types=[
            pltpu.VMEM((elems_per_worker * GRID_CELLS,), jnp.float32),
            pltpu.VMEM((512,), jnp.float32),
            pltpu.VMEM((elems_per_worker,), jnp.float32),
            pltpu.SemaphoreType.DMA,
            pltpu.SemaphoreType.DMA,
        ],
    )
    flat = grid.reshape(-1)
    return entropy_kernel(flat, jnp.asarray(_ENT_TABLE))
